# Initial kernel scaffold; baseline (speedup 1.0000x reference)
import jax
import jax.numpy as jnp
from jax.experimental import pallas as pl


def kernel(node_feature, edge_index, batch, learnable_skip, W_pre, b_pre, W0l, b0, W0r, W1l, b1, W1r, Wp1, bp1, Wp2, bp2, Wp3, bp3, Wp4, bp4):
    raise NotImplementedError("write your pallas kernel here")


# speedup vs baseline: 5.8548x; 5.8548x over previous
import functools
import jax
import jax.numpy as jnp
from jax import lax
from jax.experimental import pallas as pl
from jax.experimental.pallas import tpu as pltpu
from jax.experimental.pallas import tpu_sc as plsc

N = 50000
E = 800000
IN_DIM = 5
H = 64
NG = 64

LANES = 128
TR = 392
KB = 28
G = TR // KB
ROWS2D = 16 * TR
EP = ROWS2D * LANES

HALF = 25000
ACC_R = 25088
DUMMY = 25000
RPT = 1568

RB = 1000
GRID = N // RB


def _make_sc_scatter(with_deg: bool):
  out_type = [jax.ShapeDtypeStruct((N, H), jnp.float32)]
  if with_deg:
    out_type.append(jax.ShapeDtypeStruct((N,), jnp.float32))
  scratch = [
      pltpu.VMEM((KB, LANES), jnp.int32),
      pltpu.VMEM((KB, LANES), jnp.int32),
      pltpu.VMEM((LANES, H), jnp.float32),
      pltpu.VMEM((LANES, H), jnp.float32),
      pltpu.VMEM((LANES,), jnp.float32),
      pltpu.VMEM((112,), jnp.float32),
      pltpu.VMEM_SHARED((ACC_R, H), jnp.float32),
      pltpu.VMEM_SHARED((ACC_R,), jnp.float32),
      pltpu.SemaphoreType.DMA,
      pltpu.SemaphoreType.DMA,
  ]
  mesh = plsc.VectorSubcoreMesh(core_axis_name="c", subcore_axis_name="s")

  def body(z, srcr, dstr, *rest):
    if with_deg:
      ssum, deg_out = rest[0], rest[1]
      rest = rest[2:]
    else:
      ssum = rest[0]
      rest = rest[1:]
    (src_v, ldst_v, rows_a, rows_b, ones_v, dzero,
     acc, dacc, sem_a, sem_b) = rest
    c = lax.axis_index("c")
    s = lax.axis_index("s")
    base = c * HALF
    tb = s * RPT

    zeros16 = jnp.zeros((16,), jnp.float32)

    def zrow(r, _):
      for cc in range(H // 16):
        rows_a[r, pl.ds(cc * 16, 16)] = zeros16
      return _
    lax.fori_loop(0, LANES, zrow, None)
    for i in range(7):
      dzero[pl.ds(i * 16, 16)] = zeros16
    for i in range(LANES // 16):
      ones_v[pl.ds(i * 16, 16)] = jnp.ones((16,), jnp.float32)

    def zcp(i, _):
      pltpu.sync_copy(rows_a.at[pl.ds(0, 112)], acc.at[pl.ds(tb + i * 112, 112)])
      if with_deg:
        pltpu.sync_copy(dzero, dacc.at[pl.ds(tb + i * 112, 112)])
      return _
    lax.fori_loop(0, RPT // 112, zcp, None)
    plsc.subcore_barrier()

    row0 = s * TR

    def stage(g, _):
      rb = row0 + g * KB
      pltpu.sync_copy(srcr.at[pl.ds(rb, KB)], src_v)
      pltpu.sync_copy(dstr.at[pl.ds(rb, KB)], ldst_v)

      def ldrow(r, _2):
        for cc in range(LANES // 16):
          d = ldst_v[r, pl.ds(cc * 16, 16)]
          l = d - base
          oob = (l < 0) | (l >= HALF)
          ldst_v[r, pl.ds(cc * 16, 16)] = jnp.where(oob, DUMMY, l)
        return _2
      lax.fori_loop(0, KB, ldrow, None)

      def pair(j2, _2):
        ja = j2 * 2
        jb = ja + 1
        cp_a = pltpu.async_copy(z.at[src_v.at[ja]], rows_a, sem_a)
        cp_b = pltpu.async_copy(z.at[src_v.at[jb]], rows_b, sem_b)
        cp_a.wait()
        pltpu.sync_copy(rows_a, acc.at[ldst_v.at[ja]], add=True)
        if with_deg:
          pltpu.sync_copy(ones_v, dacc.at[ldst_v.at[ja]], add=True)
        cp_b.wait()
        pltpu.sync_copy(rows_b, acc.at[ldst_v.at[jb]], add=True)
        if with_deg:
          pltpu.sync_copy(ones_v, dacc.at[ldst_v.at[jb]], add=True)
        return _2
      lax.fori_loop(0, KB // 2, pair, None)
      return _
    lax.fori_loop(0, G, stage, None)
    plsc.subcore_barrier()

    gb = base + tb

    @pl.when(s < 15)
    def _():
      pltpu.sync_copy(acc.at[pl.ds(tb, RPT)], ssum.at[pl.ds(gb, RPT)])
      if with_deg:
        pltpu.sync_copy(dacc.at[pl.ds(tb, RPT)], deg_out.at[pl.ds(gb, RPT)])

    @pl.when(s == 15)
    def _():
      last = HALF - 15 * RPT
      pltpu.sync_copy(acc.at[pl.ds(tb, last)], ssum.at[pl.ds(gb, last)])
      if with_deg:
        pltpu.sync_copy(dacc.at[pl.ds(tb, last)], deg_out.at[pl.ds(gb, last)])

  return pl.kernel(body, out_type=out_type, mesh=mesh, scratch_types=scratch,
                   compiler_params=pltpu.CompilerParams(use_tc_tiling_on_sc=False))


_sc_scatter_deg = _make_sc_scatter(True)
_sc_scatter = _make_sc_scatter(False)


def _tc_pre_body(nf, wpre, bpre, w0l_s, x0_ref, z0_ref):
  x0 = jnp.dot(nf[...], wpre[...], preferred_element_type=jnp.float32) + bpre[...]
  x0_ref[...] = x0
  z0_ref[...] = jnp.dot(x0, w0l_s[...], preferred_element_type=jnp.float32)


def _tc_mid_body(sum0, deg, x0, b0, w0r_s, w1lt, w1lb, w1rt, w1rb,
                 x1_ref, z1_ref, r1_ref):
  rd = 1.0 / jnp.maximum(deg[...], 1.0)
  x0v = x0[...]
  x1 = jnp.maximum(sum0[...] * rd + b0[...]
                   + jnp.dot(x0v, w0r_s[...], preferred_element_type=jnp.float32), 0.0)
  x1_ref[...] = x1
  z1_ref[...] = (jnp.dot(x0v, w1lt[...], preferred_element_type=jnp.float32)
                 + jnp.dot(x1, w1lb[...], preferred_element_type=jnp.float32))
  r1_ref[...] = (jnp.dot(x0v, w1rt[...], preferred_element_type=jnp.float32)
                 + jnp.dot(x1, w1rb[...], preferred_element_type=jnp.float32))


def _tc_post_body(sum1, deg, x0, x1, r1, b1, batch,
                  wp1, bp1, wp2, bp2, wp3, bp3, wp4, bp4,
                  out_ref, pooled):
  i = pl.program_id(0)

  @pl.when(i == 0)
  def _():
    pooled[...] = jnp.zeros_like(pooled)

  rd = 1.0 / jnp.maximum(deg[...], 1.0)
  x2 = jnp.maximum(sum1[...] * rd + b1[...] + r1[...], 0.0)
  emb = jnp.concatenate([x0[...], x1[...], x2], axis=1)
  b = batch[0, 0, :]
  oh = (b[:, None] == lax.broadcasted_iota(jnp.int32, (RB, NG), 1))
  oh = oh.astype(jnp.float32)
  pooled[...] += lax.dot_general(oh, emb, (((0,), (0,)), ((), ())),
                                 preferred_element_type=jnp.float32)

  @pl.when(i == GRID - 1)
  def _():
    h = jnp.dot(pooled[...], wp1[...], preferred_element_type=jnp.float32) + bp1[...]
    h = jnp.where(h >= 0, h, 0.1 * h)
    h = jnp.maximum(jnp.dot(h, wp2[...], preferred_element_type=jnp.float32) + bp2[...], 0.0)
    h = jnp.maximum(jnp.dot(h, wp3[...], preferred_element_type=jnp.float32) + bp3[...], 0.0)
    out_ref[...] = jnp.dot(h, wp4[...], preferred_element_type=jnp.float32) + bp4[...]


def _full(shape):
  return pl.BlockSpec(shape, lambda i: (0,) * len(shape))


def _rows(width):
  return pl.BlockSpec((RB, width), lambda i: (i, 0))


@jax.jit
def kernel(node_feature, edge_index, batch, learnable_skip, W_pre, b_pre,
           W0l, b0, W0r, W1l, b1, W1r,
           Wp1, bp1, Wp2, bp2, Wp3, bp3, Wp4, bp4):
  sg = jax.nn.sigmoid(learnable_skip)
  s00 = sg[0, 0]
  s10 = sg[1, 0]
  s11 = sg[1, 1]

  src = edge_index[0]
  dst = edge_index[1]
  pad = EP - E
  srcr = jnp.concatenate([src, jnp.zeros((pad,), jnp.int32)]).reshape(ROWS2D, LANES)
  dstr = jnp.concatenate([dst, jnp.full((pad,), N, jnp.int32)]).reshape(ROWS2D, LANES)

  x0, z0 = pl.pallas_call(
      _tc_pre_body,
      grid=(GRID,),
      in_specs=[pl.BlockSpec((RB, IN_DIM), lambda i: (i, 0)),
                _full((IN_DIM, H)), _full((1, H)), _full((H, H))],
      out_specs=[_rows(H), _rows(H)],
      out_shape=[jax.ShapeDtypeStruct((N, H), jnp.float32),
                 jax.ShapeDtypeStruct((N, H), jnp.float32)],
  )(node_feature, W_pre, b_pre.reshape(1, H), s00 * W0l)

  sum0, deg = _sc_scatter_deg(z0, srcr, dstr)
  deg2 = deg.reshape(N, 1)

  x1, z1, r1 = pl.pallas_call(
      _tc_mid_body,
      grid=(GRID,),
      in_specs=[_rows(H), pl.BlockSpec((RB, 1), lambda i: (i, 0)), _rows(H),
                _full((1, H)), _full((H, H)), _full((H, H)), _full((H, H)),
                _full((H, H)), _full((H, H))],
      out_specs=[_rows(H), _rows(H), _rows(H)],
      out_shape=[jax.ShapeDtypeStruct((N, H), jnp.float32)] * 3,
  )(sum0, deg2, x0, b0.reshape(1, H), s00 * W0r,
    s10 * W1l[:H], s11 * W1l[H:], s10 * W1r[:H], s11 * W1r[H:])

  (sum1,) = _sc_scatter(z1, srcr, dstr)

  batch3 = batch.reshape(GRID, 1, RB)
  out = pl.pallas_call(
      _tc_post_body,
      grid=(GRID,),
      in_specs=[_rows(H), pl.BlockSpec((RB, 1), lambda i: (i, 0)),
                _rows(H), _rows(H), _rows(H), _full((1, H)),
                pl.BlockSpec((1, 1, RB), lambda i: (i, 0, 0)),
                _full((3 * H, H)), _full((1, H)), _full((H, H)), _full((1, H)),
                _full((H, 256)), _full((1, 256)), _full((256, H)), _full((1, H))],
      out_specs=pl.BlockSpec((NG, H), lambda i: (0, 0)),
      out_shape=jax.ShapeDtypeStruct((NG, H), jnp.float32),
      scratch_shapes=[pltpu.VMEM((NG, 3 * H), jnp.float32)],
  )(sum1, deg2, x0, x1, r1, b1.reshape(1, H), batch3,
    Wp1, bp1.reshape(1, H), Wp2, bp2.reshape(1, H),
    Wp3, bp3.reshape(1, 256), Wp4, bp4.reshape(1, H))
  return out


# speedup vs baseline: 9.5972x; 1.6392x over previous
import functools
import jax
import jax.numpy as jnp
from jax import lax
from jax.experimental import pallas as pl
from jax.experimental.pallas import tpu as pltpu
from jax.experimental.pallas import tpu_sc as plsc

N = 50000
E = 800000
IN_DIM = 5
H = 64
HW = 32
NG = 64

LANES = 128
TR = 392
KB = 28
G = TR // KB
ROWS2D = 16 * TR
EP = ROWS2D * LANES

ACC_R = 50176
RPT = 3136
NRING = 4

RB = 1000
GRID = N // RB


def _make_sc_scatter(with_deg: bool):
  out_type = [jax.ShapeDtypeStruct((N, HW), jnp.float32),
              jax.ShapeDtypeStruct((N, HW), jnp.float32)]
  if with_deg:
    out_type += [jax.ShapeDtypeStruct((N,), jnp.float32),
                 jax.ShapeDtypeStruct((N,), jnp.float32)]
  scratch = [
      pltpu.VMEM((KB, LANES), jnp.int32),
      pltpu.VMEM((KB, LANES), jnp.int32),
      [pltpu.VMEM((LANES, HW), jnp.float32)] * NRING,
      pltpu.VMEM((LANES,), jnp.float32),
      pltpu.VMEM((112,), jnp.float32),
      pltpu.VMEM_SHARED((ACC_R, HW), jnp.float32),
      pltpu.VMEM_SHARED((ACC_R,), jnp.float32),
      [pltpu.SemaphoreType.DMA] * NRING,
  ]
  mesh = plsc.VectorSubcoreMesh(core_axis_name="c", subcore_axis_name="s")

  def body(z_lo, z_hi, srcr, dstr, *rest):
    if with_deg:
      out_lo, out_hi, deg_a, deg_b = rest[:4]
      rest = rest[4:]
    else:
      out_lo, out_hi = rest[:2]
      rest = rest[2:]
    src_v, dst_v, rows, ones_v, dzero, acc, dacc, sems = rest
    c = lax.axis_index("c")
    s = lax.axis_index("s")
    tb = s * RPT

    zeros16 = jnp.zeros((16,), jnp.float32)

    def zrow(r, _):
      for cc in range(HW // 16):
        rows[0][r, pl.ds(cc * 16, 16)] = zeros16
      return _
    lax.fori_loop(0, 112, zrow, None)
    for i in range(7):
      dzero[pl.ds(i * 16, 16)] = zeros16
    for i in range(LANES // 16):
      ones_v[pl.ds(i * 16, 16)] = jnp.ones((16,), jnp.float32)

    def zcp(i, _):
      pltpu.sync_copy(rows[0].at[pl.ds(0, 112)], acc.at[pl.ds(tb + i * 112, 112)])
      if with_deg:
        pltpu.sync_copy(dzero, dacc.at[pl.ds(tb + i * 112, 112)])
      return _
    lax.fori_loop(0, RPT // 112, zcp, None)
    plsc.subcore_barrier()

    row0 = s * TR

    def stage(g, _):
      rb = row0 + g * KB
      pltpu.sync_copy(srcr.at[pl.ds(rb, KB)], src_v)
      pltpu.sync_copy(dstr.at[pl.ds(rb, KB)], dst_v)

      def quad(q, _2):
        j = q * NRING
        for k in range(NRING):
          @pl.when(c == 0)
          def _():
            pltpu.async_copy(z_lo.at[src_v.at[j + k]], rows[k], sems[k])
          @pl.when(c == 1)
          def _():
            pltpu.async_copy(z_hi.at[src_v.at[j + k]], rows[k], sems[k])
        for k in range(NRING):
          pltpu.make_async_copy(z_lo.at[src_v.at[j + k]], rows[k], sems[k]).wait()
          pltpu.sync_copy(rows[k], acc.at[dst_v.at[j + k]], add=True)
          if with_deg:
            @pl.when((g % 2) == c)
            def _():
              pltpu.sync_copy(ones_v, dacc.at[dst_v.at[j + k]], add=True)
        return _2
      lax.fori_loop(0, KB // NRING, quad, None)
      return _
    lax.fori_loop(0, G, stage, None)
    plsc.subcore_barrier()

    def wb(out_ref):
      @pl.when(s < 15)
      def _():
        pltpu.sync_copy(acc.at[pl.ds(tb, RPT)], out_ref.at[pl.ds(tb, RPT)])

      @pl.when(s == 15)
      def _():
        last = N - 15 * RPT
        pltpu.sync_copy(acc.at[pl.ds(tb, last)], out_ref.at[pl.ds(tb, last)])

    def wbd(deg_ref):
      @pl.when(s < 15)
      def _():
        pltpu.sync_copy(dacc.at[pl.ds(tb, RPT)], deg_ref.at[pl.ds(tb, RPT)])

      @pl.when(s == 15)
      def _():
        last = N - 15 * RPT
        pltpu.sync_copy(dacc.at[pl.ds(tb, last)], deg_ref.at[pl.ds(tb, last)])

    @pl.when(c == 0)
    def _():
      wb(out_lo)
    @pl.when(c == 1)
    def _():
      wb(out_hi)
    if with_deg:
      @pl.when(c == 0)
      def _():
        wbd(deg_a)
      @pl.when(c == 1)
      def _():
        wbd(deg_b)

  return pl.kernel(body, out_type=out_type, mesh=mesh, scratch_types=scratch,
                   compiler_params=pltpu.CompilerParams(use_tc_tiling_on_sc=False))


_sc_scatter_deg = _make_sc_scatter(True)
_sc_scatter = _make_sc_scatter(False)


def _tc_pre_body(nf, wpre, bpre, w0l_s, x0_ref, zlo_ref, zhi_ref):
  x0 = jnp.dot(nf[...], wpre[...], preferred_element_type=jnp.float32) + bpre[...]
  x0_ref[...] = x0
  z0 = jnp.dot(x0, w0l_s[...], preferred_element_type=jnp.float32)
  zlo_ref[...] = z0[:, :HW]
  zhi_ref[...] = z0[:, HW:]


def _tc_mid_body(sum_lo, sum_hi, deg_a, deg_b, x0, b0, w0r_s,
                 w1lt, w1lb, w1rt, w1rb,
                 x1_ref, zlo_ref, zhi_ref, r1_ref, rd_ref):
  rd = 1.0 / jnp.maximum(deg_a[...] + deg_b[...], 1.0)
  rd_ref[...] = rd
  x0v = x0[...]
  sum0 = jnp.concatenate([sum_lo[...], sum_hi[...]], axis=1)
  x1 = jnp.maximum(sum0 * rd + b0[...]
                   + jnp.dot(x0v, w0r_s[...], preferred_element_type=jnp.float32), 0.0)
  x1_ref[...] = x1
  z1 = (jnp.dot(x0v, w1lt[...], preferred_element_type=jnp.float32)
        + jnp.dot(x1, w1lb[...], preferred_element_type=jnp.float32))
  zlo_ref[...] = z1[:, :HW]
  zhi_ref[...] = z1[:, HW:]
  r1_ref[...] = (jnp.dot(x0v, w1rt[...], preferred_element_type=jnp.float32)
                 + jnp.dot(x1, w1rb[...], preferred_element_type=jnp.float32))


def _tc_post_body(sum_lo, sum_hi, rd, x0, x1, r1, b1, batch,
                  wp1, bp1, wp2, bp2, wp3, bp3, wp4, bp4,
                  out_ref, pooled):
  i = pl.program_id(0)

  @pl.when(i == 0)
  def _():
    pooled[...] = jnp.zeros_like(pooled)

  sum1 = jnp.concatenate([sum_lo[...], sum_hi[...]], axis=1)
  x2 = jnp.maximum(sum1 * rd[...] + b1[...] + r1[...], 0.0)
  emb = jnp.concatenate([x0[...], x1[...], x2], axis=1)
  b = batch[0, 0, :]
  oh = (b[:, None] == lax.broadcasted_iota(jnp.int32, (RB, NG), 1))
  oh = oh.astype(jnp.float32)
  pooled[...] += lax.dot_general(oh, emb, (((0,), (0,)), ((), ())),
                                 preferred_element_type=jnp.float32)

  @pl.when(i == GRID - 1)
  def _():
    h = jnp.dot(pooled[...], wp1[...], preferred_element_type=jnp.float32) + bp1[...]
    h = jnp.where(h >= 0, h, 0.1 * h)
    h = jnp.maximum(jnp.dot(h, wp2[...], preferred_element_type=jnp.float32) + bp2[...], 0.0)
    h = jnp.maximum(jnp.dot(h, wp3[...], preferred_element_type=jnp.float32) + bp3[...], 0.0)
    out_ref[...] = jnp.dot(h, wp4[...], preferred_element_type=jnp.float32) + bp4[...]


def _full(shape):
  return pl.BlockSpec(shape, lambda i: (0,) * len(shape))


def _rows(width):
  return pl.BlockSpec((RB, width), lambda i: (i, 0))


@jax.jit
def kernel(node_feature, edge_index, batch, learnable_skip, W_pre, b_pre,
           W0l, b0, W0r, W1l, b1, W1r,
           Wp1, bp1, Wp2, bp2, Wp3, bp3, Wp4, bp4):
  sg = jax.nn.sigmoid(learnable_skip)
  s00 = sg[0, 0]
  s10 = sg[1, 0]
  s11 = sg[1, 1]

  src = edge_index[0]
  dst = edge_index[1]
  pad = EP - E
  srcr = jnp.concatenate([src, jnp.zeros((pad,), jnp.int32)]).reshape(ROWS2D, LANES)
  dstr = jnp.concatenate([dst, jnp.full((pad,), N, jnp.int32)]).reshape(ROWS2D, LANES)

  x0, z0_lo, z0_hi = pl.pallas_call(
      _tc_pre_body,
      grid=(GRID,),
      in_specs=[pl.BlockSpec((RB, IN_DIM), lambda i: (i, 0)),
                _full((IN_DIM, H)), _full((1, H)), _full((H, H))],
      out_specs=[_rows(H), _rows(HW), _rows(HW)],
      out_shape=[jax.ShapeDtypeStruct((N, H), jnp.float32),
                 jax.ShapeDtypeStruct((N, HW), jnp.float32),
                 jax.ShapeDtypeStruct((N, HW), jnp.float32)],
  )(node_feature, W_pre, b_pre.reshape(1, H), s00 * W0l)

  sum0_lo, sum0_hi, deg_a, deg_b = _sc_scatter_deg(z0_lo, z0_hi, srcr, dstr)

  x1, z1_lo, z1_hi, r1, rd = pl.pallas_call(
      _tc_mid_body,
      grid=(GRID,),
      in_specs=[_rows(HW), _rows(HW),
                pl.BlockSpec((RB, 1), lambda i: (i, 0)),
                pl.BlockSpec((RB, 1), lambda i: (i, 0)),
                _rows(H), _full((1, H)), _full((H, H)), _full((H, H)),
                _full((H, H)), _full((H, H)), _full((H, H))],
      out_specs=[_rows(H), _rows(HW), _rows(HW), _rows(H),
                 pl.BlockSpec((RB, 1), lambda i: (i, 0))],
      out_shape=[jax.ShapeDtypeStruct((N, H), jnp.float32),
                 jax.ShapeDtypeStruct((N, HW), jnp.float32),
                 jax.ShapeDtypeStruct((N, HW), jnp.float32),
                 jax.ShapeDtypeStruct((N, H), jnp.float32),
                 jax.ShapeDtypeStruct((N, 1), jnp.float32)],
  )(sum0_lo, sum0_hi, deg_a.reshape(N, 1), deg_b.reshape(N, 1), x0,
    b0.reshape(1, H), s00 * W0r,
    s10 * W1l[:H], s11 * W1l[H:], s10 * W1r[:H], s11 * W1r[H:])

  sum1_lo, sum1_hi = _sc_scatter(z1_lo, z1_hi, srcr, dstr)

  batch3 = batch.reshape(GRID, 1, RB)
  out = pl.pallas_call(
      _tc_post_body,
      grid=(GRID,),
      in_specs=[_rows(HW), _rows(HW), pl.BlockSpec((RB, 1), lambda i: (i, 0)),
                _rows(H), _rows(H), _rows(H), _full((1, H)),
                pl.BlockSpec((1, 1, RB), lambda i: (i, 0, 0)),
                _full((3 * H, H)), _full((1, H)), _full((H, H)), _full((1, H)),
                _full((H, 256)), _full((1, 256)), _full((256, H)), _full((1, H))],
      out_specs=pl.BlockSpec((NG, H), lambda i: (0, 0)),
      out_shape=jax.ShapeDtypeStruct((NG, H), jnp.float32),
      scratch_shapes=[pltpu.VMEM((NG, 3 * H), jnp.float32)],
  )(sum1_lo, sum1_hi, rd, x0, x1, r1, b1.reshape(1, H), batch3,
    Wp1, bp1.reshape(1, H), Wp2, bp2.reshape(1, H),
    Wp3, bp3.reshape(1, 256), Wp4, bp4.reshape(1, H))
  return out


# speedup vs baseline: 11.0529x; 1.1517x over previous
import functools
import jax
import jax.numpy as jnp
from jax import lax
from jax.experimental import pallas as pl
from jax.experimental.pallas import tpu as pltpu
from jax.experimental.pallas import tpu_sc as plsc

N = 50000
E = 800000
IN_DIM = 5
H = 64
HW = 32
NG = 64

LANES = 128
TR = 392
KB = 28
G = TR // KB
ROWS2D = 16 * TR
EP = ROWS2D * LANES

ACC_R = 50176
RPT = 3136
NRING = 4

RB = 1000
GRID = N // RB


def _make_sc_scatter(with_deg: bool):
  out_type = [jax.ShapeDtypeStruct((N, HW), jnp.float32),
              jax.ShapeDtypeStruct((N, HW), jnp.float32)]
  if with_deg:
    out_type += [jax.ShapeDtypeStruct((N,), jnp.float32),
                 jax.ShapeDtypeStruct((N,), jnp.float32)]
  scratch = [
      pltpu.VMEM((KB, LANES), jnp.int32),
      pltpu.VMEM((KB, LANES), jnp.int32),
      [pltpu.VMEM((LANES, HW), jnp.float32)] * NRING,
      pltpu.VMEM((LANES,), jnp.float32),
      pltpu.VMEM((112,), jnp.float32),
      pltpu.VMEM_SHARED((ACC_R, HW), jnp.float32),
      pltpu.VMEM_SHARED((ACC_R,), jnp.float32),
      [pltpu.SemaphoreType.DMA] * NRING,
      [pltpu.SemaphoreType.DMA] * NRING,
      [pltpu.SemaphoreType.DMA] * NRING,
  ]
  mesh = plsc.VectorSubcoreMesh(core_axis_name="c", subcore_axis_name="s")

  def body(z_lo, z_hi, srcr, dstr, *rest):
    if with_deg:
      out_lo, out_hi, deg_a, deg_b = rest[:4]
      rest = rest[4:]
    else:
      out_lo, out_hi = rest[:2]
      rest = rest[2:]
    src_v, dst_v, rows, ones_v, dzero, acc, dacc, gsem, ssem, dsem = rest
    c = lax.axis_index("c")
    s = lax.axis_index("s")
    tb = s * RPT

    zeros16 = jnp.zeros((16,), jnp.float32)

    def zrow(r, _):
      for cc in range(HW // 16):
        rows[0][r, pl.ds(cc * 16, 16)] = zeros16
      return _
    lax.fori_loop(0, 112, zrow, None)
    for i in range(7):
      dzero[pl.ds(i * 16, 16)] = zeros16
    for i in range(LANES // 16):
      ones_v[pl.ds(i * 16, 16)] = jnp.ones((16,), jnp.float32)

    def zcp(i, _):
      pltpu.sync_copy(rows[0].at[pl.ds(0, 112)], acc.at[pl.ds(tb + i * 112, 112)])
      if with_deg:
        pltpu.sync_copy(dzero, dacc.at[pl.ds(tb + i * 112, 112)])
      return _
    lax.fori_loop(0, RPT // 112, zcp, None)
    plsc.subcore_barrier()

    row0 = s * TR

    def gissue(j, k):
      @pl.when(c == 0)
      def _():
        pltpu.async_copy(z_lo.at[src_v.at[j]], rows[k], gsem[k])

      @pl.when(c == 1)
      def _():
        pltpu.async_copy(z_hi.at[src_v.at[j]], rows[k], gsem[k])

    NQ = KB // NRING

    def stage(g, _):
      rb = row0 + g * KB
      pltpu.sync_copy(srcr.at[pl.ds(rb, KB)], src_v)
      pltpu.sync_copy(dstr.at[pl.ds(rb, KB)], dst_v)
      for k in range(NRING):
        gissue(k, k)

      def quad(q, _2):
        j = q * NRING
        for k in range(NRING):
          pltpu.make_async_copy(z_lo.at[src_v.at[j + k]], rows[k], gsem[k]).wait()
          pltpu.async_copy(rows[k], acc.at[dst_v.at[j + k]], ssem[k], add=True)
          if with_deg:
            @pl.when((g % 2) == c)
            def _():
              pltpu.async_copy(ones_v, dacc.at[dst_v.at[j + k]], dsem[k], add=True)
        for k in range(NRING):
          pltpu.make_async_copy(rows[k], acc.at[dst_v.at[j + k]], ssem[k]).wait()
          if with_deg:
            @pl.when((g % 2) == c)
            def _():
              pltpu.make_async_copy(ones_v, dacc.at[dst_v.at[j + k]], dsem[k]).wait()

          @pl.when(q < NQ - 1)
          def _():
            gissue(j + NRING + k, k)
        return _2
      lax.fori_loop(0, NQ, quad, None)
      return _
    lax.fori_loop(0, G, stage, None)
    plsc.subcore_barrier()

    def wb(out_ref):
      @pl.when(s < 15)
      def _():
        pltpu.sync_copy(acc.at[pl.ds(tb, RPT)], out_ref.at[pl.ds(tb, RPT)])

      @pl.when(s == 15)
      def _():
        last = N - 15 * RPT
        pltpu.sync_copy(acc.at[pl.ds(tb, last)], out_ref.at[pl.ds(tb, last)])

    def wbd(deg_ref):
      @pl.when(s < 15)
      def _():
        pltpu.sync_copy(dacc.at[pl.ds(tb, RPT)], deg_ref.at[pl.ds(tb, RPT)])

      @pl.when(s == 15)
      def _():
        last = N - 15 * RPT
        pltpu.sync_copy(dacc.at[pl.ds(tb, last)], deg_ref.at[pl.ds(tb, last)])

    @pl.when(c == 0)
    def _():
      wb(out_lo)
    @pl.when(c == 1)
    def _():
      wb(out_hi)
    if with_deg:
      @pl.when(c == 0)
      def _():
        wbd(deg_a)
      @pl.when(c == 1)
      def _():
        wbd(deg_b)

  return pl.kernel(body, out_type=out_type, mesh=mesh, scratch_types=scratch,
                   compiler_params=pltpu.CompilerParams(use_tc_tiling_on_sc=False))


_sc_scatter_deg = _make_sc_scatter(True)
_sc_scatter = _make_sc_scatter(False)


def _tc_pre_body(nf, wpre, bpre, w0l_s, x0_ref, zlo_ref, zhi_ref):
  x0 = jnp.dot(nf[...], wpre[...], preferred_element_type=jnp.float32) + bpre[...]
  x0_ref[...] = x0
  z0 = jnp.dot(x0, w0l_s[...], preferred_element_type=jnp.float32)
  zlo_ref[...] = z0[:, :HW]
  zhi_ref[...] = z0[:, HW:]


def _tc_mid_body(sum_lo, sum_hi, deg_a, deg_b, x0, b0, w0r_s,
                 w1lt, w1lb, w1rt, w1rb,
                 x1_ref, zlo_ref, zhi_ref, r1_ref, rd_ref):
  rd = 1.0 / jnp.maximum(deg_a[...] + deg_b[...], 1.0)
  rd_ref[...] = rd
  x0v = x0[...]
  sum0 = jnp.concatenate([sum_lo[...], sum_hi[...]], axis=1)
  x1 = jnp.maximum(sum0 * rd + b0[...]
                   + jnp.dot(x0v, w0r_s[...], preferred_element_type=jnp.float32), 0.0)
  x1_ref[...] = x1
  z1 = (jnp.dot(x0v, w1lt[...], preferred_element_type=jnp.float32)
        + jnp.dot(x1, w1lb[...], preferred_element_type=jnp.float32))
  zlo_ref[...] = z1[:, :HW]
  zhi_ref[...] = z1[:, HW:]
  r1_ref[...] = (jnp.dot(x0v, w1rt[...], preferred_element_type=jnp.float32)
                 + jnp.dot(x1, w1rb[...], preferred_element_type=jnp.float32))


def _tc_post_body(sum_lo, sum_hi, rd, x0, x1, r1, b1, batch,
                  wp1, bp1, wp2, bp2, wp3, bp3, wp4, bp4,
                  out_ref, pooled):
  i = pl.program_id(0)

  @pl.when(i == 0)
  def _():
    pooled[...] = jnp.zeros_like(pooled)

  sum1 = jnp.concatenate([sum_lo[...], sum_hi[...]], axis=1)
  x2 = jnp.maximum(sum1 * rd[...] + b1[...] + r1[...], 0.0)
  emb = jnp.concatenate([x0[...], x1[...], x2], axis=1)
  b = batch[0, 0, :]
  oh = (b[:, None] == lax.broadcasted_iota(jnp.int32, (RB, NG), 1))
  oh = oh.astype(jnp.float32)
  pooled[...] += lax.dot_general(oh, emb, (((0,), (0,)), ((), ())),
                                 preferred_element_type=jnp.float32)

  @pl.when(i == GRID - 1)
  def _():
    h = jnp.dot(pooled[...], wp1[...], preferred_element_type=jnp.float32) + bp1[...]
    h = jnp.where(h >= 0, h, 0.1 * h)
    h = jnp.maximum(jnp.dot(h, wp2[...], preferred_element_type=jnp.float32) + bp2[...], 0.0)
    h = jnp.maximum(jnp.dot(h, wp3[...], preferred_element_type=jnp.float32) + bp3[...], 0.0)
    out_ref[...] = jnp.dot(h, wp4[...], preferred_element_type=jnp.float32) + bp4[...]


def _full(shape):
  return pl.BlockSpec(shape, lambda i: (0,) * len(shape))


def _rows(width):
  return pl.BlockSpec((RB, width), lambda i: (i, 0))


@jax.jit
def kernel(node_feature, edge_index, batch, learnable_skip, W_pre, b_pre,
           W0l, b0, W0r, W1l, b1, W1r,
           Wp1, bp1, Wp2, bp2, Wp3, bp3, Wp4, bp4):
  sg = jax.nn.sigmoid(learnable_skip)
  s00 = sg[0, 0]
  s10 = sg[1, 0]
  s11 = sg[1, 1]

  src = edge_index[0]
  dst = edge_index[1]
  pad = EP - E
  srcr = jnp.concatenate([src, jnp.zeros((pad,), jnp.int32)]).reshape(ROWS2D, LANES)
  dstr = jnp.concatenate([dst, jnp.full((pad,), N, jnp.int32)]).reshape(ROWS2D, LANES)

  x0, z0_lo, z0_hi = pl.pallas_call(
      _tc_pre_body,
      grid=(GRID,),
      in_specs=[pl.BlockSpec((RB, IN_DIM), lambda i: (i, 0)),
                _full((IN_DIM, H)), _full((1, H)), _full((H, H))],
      out_specs=[_rows(H), _rows(HW), _rows(HW)],
      out_shape=[jax.ShapeDtypeStruct((N, H), jnp.float32),
                 jax.ShapeDtypeStruct((N, HW), jnp.float32),
                 jax.ShapeDtypeStruct((N, HW), jnp.float32)],
  )(node_feature, W_pre, b_pre.reshape(1, H), s00 * W0l)

  sum0_lo, sum0_hi, deg_a, deg_b = _sc_scatter_deg(z0_lo, z0_hi, srcr, dstr)

  x1, z1_lo, z1_hi, r1, rd = pl.pallas_call(
      _tc_mid_body,
      grid=(GRID,),
      in_specs=[_rows(HW), _rows(HW),
                pl.BlockSpec((RB, 1), lambda i: (i, 0)),
                pl.BlockSpec((RB, 1), lambda i: (i, 0)),
                _rows(H), _full((1, H)), _full((H, H)), _full((H, H)),
                _full((H, H)), _full((H, H)), _full((H, H))],
      out_specs=[_rows(H), _rows(HW), _rows(HW), _rows(H),
                 pl.BlockSpec((RB, 1), lambda i: (i, 0))],
      out_shape=[jax.ShapeDtypeStruct((N, H), jnp.float32),
                 jax.ShapeDtypeStruct((N, HW), jnp.float32),
                 jax.ShapeDtypeStruct((N, HW), jnp.float32),
                 jax.ShapeDtypeStruct((N, H), jnp.float32),
                 jax.ShapeDtypeStruct((N, 1), jnp.float32)],
  )(sum0_lo, sum0_hi, deg_a.reshape(N, 1), deg_b.reshape(N, 1), x0,
    b0.reshape(1, H), s00 * W0r,
    s10 * W1l[:H], s11 * W1l[H:], s10 * W1r[:H], s11 * W1r[H:])

  sum1_lo, sum1_hi = _sc_scatter(z1_lo, z1_hi, srcr, dstr)

  batch3 = batch.reshape(GRID, 1, RB)
  out = pl.pallas_call(
      _tc_post_body,
      grid=(GRID,),
      in_specs=[_rows(HW), _rows(HW), pl.BlockSpec((RB, 1), lambda i: (i, 0)),
                _rows(H), _rows(H), _rows(H), _full((1, H)),
                pl.BlockSpec((1, 1, RB), lambda i: (i, 0, 0)),
                _full((3 * H, H)), _full((1, H)), _full((H, H)), _full((1, H)),
                _full((H, 256)), _full((1, 256)), _full((256, H)), _full((1, H))],
      out_specs=pl.BlockSpec((NG, H), lambda i: (0, 0)),
      out_shape=jax.ShapeDtypeStruct((NG, H), jnp.float32),
      scratch_shapes=[pltpu.VMEM((NG, 3 * H), jnp.float32)],
  )(sum1_lo, sum1_hi, rd, x0, x1, r1, b1.reshape(1, H), batch3,
    Wp1, bp1.reshape(1, H), Wp2, bp2.reshape(1, H),
    Wp3, bp3.reshape(1, 256), Wp4, bp4.reshape(1, H))
  return out


# speedup vs baseline: 12.5336x; 1.1340x over previous
import functools
import jax
import jax.numpy as jnp
from jax import lax
from jax.experimental import pallas as pl
from jax.experimental.pallas import tpu as pltpu
from jax.experimental.pallas import tpu_sc as plsc

N = 50000
E = 800000
IN_DIM = 5
H = 64
HW = 32
NG = 64
NP = 50048

LANES = 128
TR = 392
KB = 28
G = TR // KB
ROWS2D = 16 * TR
EP = ROWS2D * LANES

ACC_R = 50176
RPT = 3136
NRING = 4

RB = 1000
GRID = N // RB


def _deg_pieces(t):
  s0 = t * RPT
  e0 = min(s0 + RPT, N)
  pieces = []
  p = s0
  while p < e0:
    row, col = divmod(p, RB)
    ln = min(RB - col, e0 - p)
    pieces.append((p - s0, row, col, ln))
    p += ln
  return pieces


def _make_sc_scatter(with_deg: bool):
  out_type = [jax.ShapeDtypeStruct((N, H), jnp.float32)]
  if with_deg:
    out_type += [jax.ShapeDtypeStruct((GRID, 1, RB), jnp.float32),
                 jax.ShapeDtypeStruct((GRID, 1, RB), jnp.float32)]
  scratch = [
      pltpu.VMEM((KB, LANES), jnp.int32),
      pltpu.VMEM((KB, LANES), jnp.int32),
      [pltpu.VMEM((LANES, HW), jnp.float32)] * NRING,
      pltpu.VMEM((LANES,), jnp.float32),
      pltpu.VMEM((112,), jnp.float32),
      pltpu.VMEM_SHARED((ACC_R, HW), jnp.float32),
      pltpu.VMEM_SHARED((ACC_R,), jnp.float32),
      [pltpu.SemaphoreType.DMA] * NRING,
      [pltpu.SemaphoreType.DMA] * NRING,
      [pltpu.SemaphoreType.DMA] * NRING,
  ]
  mesh = plsc.VectorSubcoreMesh(core_axis_name="c", subcore_axis_name="s")

  def body(z_lo, z_hi, ei3, *rest):
    if with_deg:
      ssum, deg_a, deg_b = rest[:3]
      rest = rest[3:]
    else:
      ssum = rest[0]
      rest = rest[1:]
    src_v, dst_v, rows, ones_v, dzero, acc, dacc, gsem, ssem, dsem = rest
    c = lax.axis_index("c")
    s = lax.axis_index("s")
    tb = s * RPT

    zeros16 = jnp.zeros((16,), jnp.float32)

    def zrow(r, _):
      for cc in range(HW // 16):
        rows[0][r, pl.ds(cc * 16, 16)] = zeros16
      return _
    lax.fori_loop(0, 112, zrow, None)
    for i in range(7):
      dzero[pl.ds(i * 16, 16)] = zeros16
    for i in range(LANES // 16):
      ones_v[pl.ds(i * 16, 16)] = jnp.ones((16,), jnp.float32)

    def zcp(i, _):
      pltpu.sync_copy(rows[0].at[pl.ds(0, 112)], acc.at[pl.ds(tb + i * 112, 112)])
      if with_deg:
        pltpu.sync_copy(dzero, dacc.at[pl.ds(tb + i * 112, 112)])
      return _
    lax.fori_loop(0, RPT // 112, zcp, None)
    plsc.subcore_barrier()

    row0 = s * TR

    def gissue(j, k):
      @pl.when(c == 0)
      def _():
        pltpu.async_copy(z_lo.at[src_v.at[j]], rows[k], gsem[k])

      @pl.when(c == 1)
      def _():
        pltpu.async_copy(z_hi.at[src_v.at[j]], rows[k], gsem[k])

    NQ = KB // NRING

    def stage(g, _):
      rb = row0 + g * KB
      pltpu.sync_copy(ei3.at[0, pl.ds(rb, KB)], src_v)
      pltpu.sync_copy(ei3.at[1, pl.ds(rb, KB)], dst_v)
      for k in range(NRING):
        gissue(k, k)

      def quad(q, _2):
        j = q * NRING
        for k in range(NRING):
          pltpu.make_async_copy(z_lo.at[src_v.at[j + k]], rows[k], gsem[k]).wait()
          pltpu.async_copy(rows[k], acc.at[dst_v.at[j + k]], ssem[k], add=True)
          if with_deg:
            @pl.when((g % 2) == c)
            def _():
              pltpu.async_copy(ones_v, dacc.at[dst_v.at[j + k]], dsem[k], add=True)
        for k in range(NRING):
          pltpu.make_async_copy(rows[k], acc.at[dst_v.at[j + k]], ssem[k]).wait()
          if with_deg:
            @pl.when((g % 2) == c)
            def _():
              pltpu.make_async_copy(ones_v, dacc.at[dst_v.at[j + k]], dsem[k]).wait()

          @pl.when(q < NQ - 1)
          def _():
            gissue(j + NRING + k, k)
        return _2
      lax.fori_loop(0, NQ, quad, None)
      return _
    lax.fori_loop(0, G, stage, None)
    plsc.subcore_barrier()

    def wb(col):
      @pl.when(s < 15)
      def _():
        pltpu.sync_copy(acc.at[pl.ds(tb, RPT)],
                        ssum.at[pl.ds(tb, RPT), pl.ds(col, HW)])

      @pl.when(s == 15)
      def _():
        last = N - 15 * RPT
        pltpu.sync_copy(acc.at[pl.ds(tb, last)],
                        ssum.at[pl.ds(tb, last), pl.ds(col, HW)])

    @pl.when(c == 0)
    def _():
      wb(0)
    @pl.when(c == 1)
    def _():
      wb(HW)

    if with_deg:
      for t in range(16):
        @pl.when(s == t)
        def _():
          for off, row, col, ln in _deg_pieces(t):
            @pl.when(c == 0)
            def _():
              pltpu.sync_copy(dacc.at[pl.ds(t * RPT + off, ln)],
                              deg_a.at[row, 0, pl.ds(col, ln)])
            @pl.when(c == 1)
            def _():
              pltpu.sync_copy(dacc.at[pl.ds(t * RPT + off, ln)],
                              deg_b.at[row, 0, pl.ds(col, ln)])

  return pl.kernel(body, out_type=out_type, mesh=mesh, scratch_types=scratch,
                   compiler_params=pltpu.CompilerParams(use_tc_tiling_on_sc=False))


_sc_scatter_deg = _make_sc_scatter(True)
_sc_scatter = _make_sc_scatter(False)


def _tc_pre_body(nf, wpre, bpre, w0l_s, x0_ref, zlo_ref, zhi_ref):
  x0 = jnp.dot(nf[...], wpre[...], preferred_element_type=jnp.float32) + bpre[...]
  x0_ref[...] = x0
  z0 = jnp.dot(x0, w0l_s[...], preferred_element_type=jnp.float32)
  zlo_ref[...] = z0[:, :HW]
  zhi_ref[...] = z0[:, HW:]


def _tc_mid_body(sum0, deg_a, deg_b, x0, b0, w0r_s,
                 w1lt, w1lb, w1rt, w1rb,
                 x1_ref, zlo_ref, zhi_ref, r1_ref, rd_ref):
  rd1 = 1.0 / jnp.maximum(deg_a[0, 0, :] + deg_b[0, 0, :], 1.0)
  rd_ref[0, 0, :] = rd1
  rd = rd1[:, None]
  x0v = x0[...]
  x1 = jnp.maximum(sum0[...] * rd + b0[...]
                   + jnp.dot(x0v, w0r_s[...], preferred_element_type=jnp.float32), 0.0)
  x1_ref[...] = x1
  z1 = (jnp.dot(x0v, w1lt[...], preferred_element_type=jnp.float32)
        + jnp.dot(x1, w1lb[...], preferred_element_type=jnp.float32))
  zlo_ref[...] = z1[:, :HW]
  zhi_ref[...] = z1[:, HW:]
  r1_ref[...] = (jnp.dot(x0v, w1rt[...], preferred_element_type=jnp.float32)
                 + jnp.dot(x1, w1rb[...], preferred_element_type=jnp.float32))


def _tc_post_body(sum1, rd3, x0, x1, r1, b1, batch,
                  wp1, bp1, wp2, bp2, wp3, bp3, wp4, bp4,
                  out_ref, pooled):
  i = pl.program_id(0)

  @pl.when(i == 0)
  def _():
    pooled[...] = jnp.zeros_like(pooled)

  rd = rd3[0, 0, :][:, None]
  x2 = jnp.maximum(sum1[...] * rd + b1[...] + r1[...], 0.0)
  emb = jnp.concatenate([x0[...], x1[...], x2], axis=1)
  b = batch[0, 0, :]
  oh = (b[:, None] == lax.broadcasted_iota(jnp.int32, (RB, NG), 1))
  oh = oh.astype(jnp.float32)
  pooled[...] += lax.dot_general(oh, emb, (((0,), (0,)), ((), ())),
                                 preferred_element_type=jnp.float32)

  @pl.when(i == GRID - 1)
  def _():
    h = jnp.dot(pooled[...], wp1[...], preferred_element_type=jnp.float32) + bp1[...]
    h = jnp.where(h >= 0, h, 0.1 * h)
    h = jnp.maximum(jnp.dot(h, wp2[...], preferred_element_type=jnp.float32) + bp2[...], 0.0)
    h = jnp.maximum(jnp.dot(h, wp3[...], preferred_element_type=jnp.float32) + bp3[...], 0.0)
    out_ref[...] = jnp.dot(h, wp4[...], preferred_element_type=jnp.float32) + bp4[...]


def _full(shape):
  return pl.BlockSpec(shape, lambda i: (0,) * len(shape))


def _rows(width):
  return pl.BlockSpec((RB, width), lambda i: (i, 0))


_VEC3 = pl.BlockSpec((1, 1, RB), lambda i: (i, 0, 0))


@jax.jit
def kernel(node_feature, edge_index, batch, learnable_skip, W_pre, b_pre,
           W0l, b0, W0r, W1l, b1, W1r,
           Wp1, bp1, Wp2, bp2, Wp3, bp3, Wp4, bp4):
  sg = jax.nn.sigmoid(learnable_skip)
  s00 = sg[0, 0]
  s10 = sg[1, 0]
  s11 = sg[1, 1]

  ei3 = jnp.pad(edge_index, ((0, 0), (0, EP - E)),
                constant_values=N).reshape(2, ROWS2D, LANES)

  x0, z0_lo, z0_hi = pl.pallas_call(
      _tc_pre_body,
      grid=(GRID,),
      in_specs=[pl.BlockSpec((RB, IN_DIM), lambda i: (i, 0)),
                _full((IN_DIM, H)), _full((1, H)), _full((H, H))],
      out_specs=[_rows(H), _rows(HW), _rows(HW)],
      out_shape=[jax.ShapeDtypeStruct((N, H), jnp.float32),
                 jax.ShapeDtypeStruct((NP, HW), jnp.float32),
                 jax.ShapeDtypeStruct((NP, HW), jnp.float32)],
  )(node_feature, W_pre, b_pre.reshape(1, H), s00 * W0l)

  sum0, deg_a, deg_b = _sc_scatter_deg(z0_lo, z0_hi, ei3)

  x1, z1_lo, z1_hi, r1, rd3 = pl.pallas_call(
      _tc_mid_body,
      grid=(GRID,),
      in_specs=[_rows(H), _VEC3, _VEC3,
                _rows(H), _full((1, H)), _full((H, H)), _full((H, H)),
                _full((H, H)), _full((H, H)), _full((H, H))],
      out_specs=[_rows(H), _rows(HW), _rows(HW), _rows(H), _VEC3],
      out_shape=[jax.ShapeDtypeStruct((N, H), jnp.float32),
                 jax.ShapeDtypeStruct((NP, HW), jnp.float32),
                 jax.ShapeDtypeStruct((NP, HW), jnp.float32),
                 jax.ShapeDtypeStruct((N, H), jnp.float32),
                 jax.ShapeDtypeStruct((GRID, 1, RB), jnp.float32)],
  )(sum0, deg_a, deg_b, x0, b0.reshape(1, H), s00 * W0r,
    s10 * W1l[:H], s11 * W1l[H:], s10 * W1r[:H], s11 * W1r[H:])

  (sum1,) = _sc_scatter(z1_lo, z1_hi, ei3)

  batch3 = batch.reshape(GRID, 1, RB)
  out = pl.pallas_call(
      _tc_post_body,
      grid=(GRID,),
      in_specs=[_rows(H), _VEC3,
                _rows(H), _rows(H), _rows(H), _full((1, H)),
                _VEC3,
                _full((3 * H, H)), _full((1, H)), _full((H, H)), _full((1, H)),
                _full((H, 256)), _full((1, 256)), _full((256, H)), _full((1, H))],
      out_specs=pl.BlockSpec((NG, H), lambda i: (0, 0)),
      out_shape=jax.ShapeDtypeStruct((NG, H), jnp.float32),
      scratch_shapes=[pltpu.VMEM((NG, 3 * H), jnp.float32)],
  )(sum1, rd3, x0, x1, r1, b1.reshape(1, H), batch3,
    Wp1, bp1.reshape(1, H), Wp2, bp2.reshape(1, H),
    Wp3, bp3.reshape(1, 256), Wp4, bp4.reshape(1, H))
  return out
